# baseline (speedup 1.0000x reference)
import functools

import jax
import jax.numpy as jnp
from jax import lax
from jax.experimental import pallas as pl
from jax.experimental.pallas import tpu as pltpu
from jax.experimental.pallas import tpu_sc as plsc

_TEMP = 0.07
_B = 4096
_D = 16
_K = 5
_NC, _NS, _L = 2, 16, 16
_NW = _NC * _NS
_CW = _B // _NW
_CH = 128
_NCH = _B // _CH
_NG = _CW // _L
_GI = 4
_RB = 256
_NEG = -3.0e38


def _norm_rows(x):
    return x * lax.rsqrt(jnp.maximum(jnp.sum(x * x, axis=1, keepdims=True),
                                     1e-24))


def _norm_body(v_ref, a_ref, vn_ref, an_ref, pos_ref):
    s = 1.0 / (_TEMP ** 0.5)
    vn = _norm_rows(v_ref[...]) * s
    an = _norm_rows(a_ref[...]) * s
    vn_ref[...] = vn.astype(jnp.bfloat16)
    an_ref[...] = an.astype(jnp.bfloat16)
    pos_ref[...] = jnp.sum(vn * an, axis=1, keepdims=True)


def _normalize(v_emb, a_emb):
    return pl.pallas_call(
        _norm_body,
        out_shape=[
            jax.ShapeDtypeStruct((_B, _D), jnp.bfloat16),
            jax.ShapeDtypeStruct((_B, _D), jnp.bfloat16),
            jax.ShapeDtypeStruct((_B, 1), jnp.float32),
        ],
    )(v_emb, a_emb)


def _sim_body(lhs_ref, rhs_ref, m_ref):
    i = pl.program_id(0)
    blk = lax.dot_general(lhs_ref[...], rhs_ref[...], (((1,), (1,)), ((), ())),
                          preferred_element_type=jnp.float32)
    m_ref[...] = blk.astype(jnp.bfloat16)
    eye = (lax.broadcasted_iota(jnp.int32, (_RB, _RB), 0)
           == lax.broadcasted_iota(jnp.int32, (_RB, _RB), 1))
    sub = m_ref[:, pl.ds(i * _RB, _RB)]
    m_ref[:, pl.ds(i * _RB, _RB)] = jnp.where(
        eye, jnp.bfloat16(_NEG), sub)


def _compute_sim(lhs, rhs):
    return pl.pallas_call(
        _sim_body,
        grid=(_B // _RB,),
        in_specs=[
            pl.BlockSpec((_RB, _D), lambda i: (i, 0)),
            pl.BlockSpec((_B, _D), lambda i: (0, 0)),
        ],
        out_specs=pl.BlockSpec((_RB, _B), lambda i: (i, 0)),
        out_shape=jax.ShapeDtypeStruct((_B, _B), jnp.bfloat16),
    )(lhs, rhs)


def _topk_body(m_hbm, out_hbm, chunk_v, acc_v, stage_v, sem0, sem1):
    wid = lax.axis_index("s") * _NC + lax.axis_index("c")
    cb = wid * _CW
    neg = jnp.full((2, _L), _NEG, jnp.bfloat16)
    sems = (sem0, sem1)

    for g in range(_NG):
        for t in range(_K):
            acc_v[g, t, :, :] = neg

    for b in range(2):
        pltpu.async_copy(
            m_hbm.at[pl.ds(b * _CH, _CH), pl.ds(cb, _CW)],
            chunk_v.at[b], sems[b])

    @pl.loop(0, _NCH, step=2)
    def _(ci0):
        for b in range(2):
            ci = ci0 + b
            pltpu.make_async_copy(
                m_hbm.at[pl.ds(0, _CH), pl.ds(cb, _CW)],
                chunk_v.at[b], sems[b]).wait()
            for gg in range(_NG // _GI):
                groups = [gg * _GI + q for q in range(_GI)]
                ts = tuple(acc_v[g, t, :, :]
                           for g in groups for t in range(_K))

                def jblock(jj, ts, groups=groups, b=b):
                    j = pl.multiple_of(2 * jj, 2)
                    out_all = []
                    for q, g in enumerate(groups):
                        cur = chunk_v[b, pl.ds(j, 2), pl.ds(g * _L, _L)]
                        ts_g = list(ts[q * _K:(q + 1) * _K])
                        for t in range(_K):
                            nt = jnp.maximum(ts_g[t], cur)
                            cur = jnp.minimum(ts_g[t], cur)
                            ts_g[t] = nt
                        out_all.extend(ts_g)
                    return tuple(out_all)

                ts = lax.fori_loop(0, _CH // 2, jblock, ts)
                for q, g in enumerate(groups):
                    for t in range(_K):
                        acc_v[g, t, :, :] = ts[q * _K + t]

            nci = ci + 2

            @pl.when(nci < _NCH)
            def _(b=b, nci=nci):
                pltpu.async_copy(
                    m_hbm.at[pl.ds(nci * _CH, _CH), pl.ds(cb, _CW)],
                    chunk_v.at[b], sems[b])

    for t in range(_K):
        for g in range(_NG):
            stage_v[t, :, pl.ds(g * _L, _L)] = acc_v[g, t, :, :]
    pltpu.sync_copy(stage_v, out_hbm.at[wid])


@functools.cache
def _topk_sc():
    return pl.kernel(
        _topk_body,
        out_type=jax.ShapeDtypeStruct((_NW, _K, 2, _CW), jnp.bfloat16),
        mesh=plsc.VectorSubcoreMesh(core_axis_name="c", subcore_axis_name="s",
                                    num_cores=_NC, num_subcores=_NS),
        scratch_types=[
            pltpu.VMEM((2, _CH, _CW), jnp.bfloat16),
            pltpu.VMEM((_NG, _K, 2, _L), jnp.bfloat16),
            pltpu.VMEM((_K, 2, _CW), jnp.bfloat16),
            pltpu.SemaphoreType.DMA,
            pltpu.SemaphoreType.DMA,
        ],
    )


def _reduce_body(s_ref, pos_ref, o_ref):
    x = s_ref[...].astype(jnp.float32)
    p32 = pos_ref[...]
    p = jnp.concatenate([p32, p32], axis=0)
    s = jnp.zeros((2 * _NW, 1, _CW), jnp.float32)
    ii = lax.broadcasted_iota(jnp.int32, x.shape, 1)
    for _ in range(_K):
        mx = jnp.max(x, axis=1, keepdims=True)
        is_mx = x == mx
        mn_i = jnp.min(jnp.where(is_mx, ii, 2 * _K), axis=1, keepdims=True)
        s = s + jnp.exp(mx - p)
        x = jnp.where(ii == mn_i, _NEG, x)
    o_ref[0, 0] = jnp.sum(jnp.log1p(s)) * (1.0 / (2 * _B))


def _reduce(s10, pos3):
    out = pl.pallas_call(
        _reduce_body,
        out_specs=pl.BlockSpec(memory_space=pltpu.SMEM),
        out_shape=jax.ShapeDtypeStruct((1, 1), jnp.float32),
    )(s10, pos3)
    return out[0, 0]


def kernel(v_emb, a_emb):
    vn, an, pos = _normalize(v_emb, a_emb)
    mt = _compute_sim(an, vn)
    s10_t = _topk_sc()(mt)
    ms = _compute_sim(vn, an)
    s10_s = _topk_sc()(ms)
    s10 = jnp.concatenate([s10_t, s10_s], axis=0).reshape(2 * _NW, 2 * _K, _CW)
    return _reduce(s10, pos.reshape(_NW, 1, _CW))


# speedup vs baseline: 1.0204x; 1.0204x over previous
import functools

import jax
import jax.numpy as jnp
from jax import lax
from jax.experimental import pallas as pl
from jax.experimental.pallas import tpu as pltpu
from jax.experimental.pallas import tpu_sc as plsc

_TEMP = 0.07
_B = 4096
_D = 16
_K = 5
_NC, _NS, _L = 2, 16, 16
_NW = _NC * _NS
_CW = 256
_RH = _B // 2
_CH = 128
_NCH = _RH // _CH
_NG = _CW // _L
_GI = 4
_JU = 2
_RB = 256
_NEG = -3.0e38


def _norm_rows(x):
    return x * lax.rsqrt(jnp.maximum(jnp.sum(x * x, axis=1, keepdims=True),
                                     1e-24))


def _norm_body(v_ref, a_ref, vn_ref, an_ref, pos_ref):
    s = 1.0 / (_TEMP ** 0.5)
    vn = _norm_rows(v_ref[...]) * s
    an = _norm_rows(a_ref[...]) * s
    vn_ref[...] = vn.astype(jnp.bfloat16)
    an_ref[...] = an.astype(jnp.bfloat16)
    pos_ref[...] = jnp.sum(vn * an, axis=1, keepdims=True)


def _normalize(v_emb, a_emb):
    return pl.pallas_call(
        _norm_body,
        out_shape=[
            jax.ShapeDtypeStruct((_B, _D), jnp.bfloat16),
            jax.ShapeDtypeStruct((_B, _D), jnp.bfloat16),
            jax.ShapeDtypeStruct((_B, 1), jnp.float32),
        ],
    )(v_emb, a_emb)


def _sim_body(lhs_ref, rhs_ref, m_ref):
    i = pl.program_id(0)
    blk = lax.dot_general(lhs_ref[...], rhs_ref[...], (((1,), (1,)), ((), ())),
                          preferred_element_type=jnp.float32)
    m_ref[...] = blk.astype(jnp.bfloat16)
    eye = (lax.broadcasted_iota(jnp.int32, (_RB, _RB), 0)
           == lax.broadcasted_iota(jnp.int32, (_RB, _RB), 1))
    sub = m_ref[:, pl.ds(i * _RB, _RB)]
    m_ref[:, pl.ds(i * _RB, _RB)] = jnp.where(
        eye, jnp.bfloat16(_NEG), sub)


def _compute_sim(lhs, rhs):
    return pl.pallas_call(
        _sim_body,
        grid=(_B // _RB,),
        in_specs=[
            pl.BlockSpec((_RB, _D), lambda i: (i, 0)),
            pl.BlockSpec((_B, _D), lambda i: (0, 0)),
        ],
        out_specs=pl.BlockSpec((_RB, _B), lambda i: (i, 0)),
        out_shape=jax.ShapeDtypeStruct((_B, _B), jnp.bfloat16),
    )(lhs, rhs)


def _topk_body(m_hbm, out_hbm, chunk_v, acc_v, stage_v, sem0, sem1):
    wid = lax.axis_index("s") * _NC + lax.axis_index("c")
    rw = wid & 1
    w2 = wid >> 1
    cb = w2 * _CW
    r0 = rw * _RH
    neg = jnp.full((2, _L), _NEG, jnp.bfloat16)
    sems = (sem0, sem1)

    for g in range(_NG):
        for t in range(_K):
            acc_v[g, t, :, :] = neg

    for b in range(2):
        pltpu.async_copy(
            m_hbm.at[pl.ds(r0 + b * _CH, _CH), pl.ds(cb, _CW)],
            chunk_v.at[b], sems[b])

    @pl.loop(0, _NCH, step=2)
    def _(ci0):
        for b in range(2):
            ci = ci0 + b
            pltpu.make_async_copy(
                m_hbm.at[pl.ds(0, _CH), pl.ds(cb, _CW)],
                chunk_v.at[b], sems[b]).wait()
            for gg in range(_NG // _GI):
                groups = [gg * _GI + q for q in range(_GI)]
                ts = tuple(acc_v[g, t, :, :]
                           for g in groups for t in range(_K))

                def jblock(jj, ts, groups=groups, b=b):
                    j = pl.multiple_of(2 * _JU * jj, 2)
                    out_all = []
                    for q, g in enumerate(groups):
                        ts_g = list(ts[q * _K:(q + 1) * _K])
                        for u in range(_JU):
                            cur = chunk_v[b, pl.ds(j + 2 * u, 2),
                                          pl.ds(g * _L, _L)]
                            for t in range(_K):
                                nt = jnp.maximum(ts_g[t], cur)
                                cur = jnp.minimum(ts_g[t], cur)
                                ts_g[t] = nt
                        out_all.extend(ts_g)
                    return tuple(out_all)

                ts = lax.fori_loop(0, _CH // (2 * _JU), jblock, ts)
                for q, g in enumerate(groups):
                    for t in range(_K):
                        acc_v[g, t, :, :] = ts[q * _K + t]

            nci = ci + 2

            @pl.when(nci < _NCH)
            def _(b=b, nci=nci):
                pltpu.async_copy(
                    m_hbm.at[pl.ds(r0 + nci * _CH, _CH), pl.ds(cb, _CW)],
                    chunk_v.at[b], sems[b])

    for t in range(_K):
        for g in range(_NG):
            stage_v[t, :, pl.ds(g * _L, _L)] = acc_v[g, t, :, :]
    pltpu.sync_copy(stage_v, out_hbm.at[wid])


@functools.cache
def _topk_sc():
    return pl.kernel(
        _topk_body,
        out_type=jax.ShapeDtypeStruct((_NW, _K, 2, _CW), jnp.bfloat16),
        mesh=plsc.VectorSubcoreMesh(core_axis_name="c", subcore_axis_name="s",
                                    num_cores=_NC, num_subcores=_NS),
        scratch_types=[
            pltpu.VMEM((2, _CH, _CW), jnp.bfloat16),
            pltpu.VMEM((_NG, _K, 2, _L), jnp.bfloat16),
            pltpu.VMEM((_K, 2, _CW), jnp.bfloat16),
            pltpu.SemaphoreType.DMA,
            pltpu.SemaphoreType.DMA,
        ],
    )


def _reduce_body(s_ref, pos_ref, o_ref):
    x = s_ref[...].astype(jnp.float32)
    p16 = pos_ref[...]
    p = jnp.concatenate([p16, p16], axis=0)
    s = jnp.zeros((x.shape[0], 1, _CW), jnp.float32)
    ii = lax.broadcasted_iota(jnp.int32, x.shape, 1)
    for _ in range(_K):
        mx = jnp.max(x, axis=1, keepdims=True)
        is_mx = x == mx
        mn_i = jnp.min(jnp.where(is_mx, ii, 4 * _K), axis=1, keepdims=True)
        s = s + jnp.exp(mx - p)
        x = jnp.where(ii == mn_i, _NEG, x)
    o_ref[0, 0] = jnp.sum(jnp.log1p(s)) * (1.0 / (2 * _B))


def _reduce(s10, pos3):
    out = pl.pallas_call(
        _reduce_body,
        out_specs=pl.BlockSpec(memory_space=pltpu.SMEM),
        out_shape=jax.ShapeDtypeStruct((1, 1), jnp.float32),
    )(s10, pos3)
    return out[0, 0]


def kernel(v_emb, a_emb):
    vn, an, pos = _normalize(v_emb, a_emb)
    mt = _compute_sim(an, vn)
    s_t = _topk_sc()(mt).reshape(_NW // 2, 4 * _K, _CW)
    ms = _compute_sim(vn, an)
    s_s = _topk_sc()(ms).reshape(_NW // 2, 4 * _K, _CW)
    s20 = jnp.concatenate([s_t, s_s], axis=0)
    return _reduce(s20, pos.reshape(_NW // 2, 1, _CW))
